# baseline (speedup 1.0000x reference)
import functools

import jax
import jax.numpy as jnp
from jax import lax
from jax.experimental import pallas as pl
from jax.experimental.pallas import tpu as pltpu
from jax.experimental.pallas import tpu_sc as plsc

SCALE = 8.0


def kernel(X, table):
    R, S = X.shape
    V, D = table.shape
    RB = R // 128
    n_tasks = S * RB

    info = plsc.get_sparse_core_info()
    NC, NS = info.num_cores, info.num_subcores
    NW = NC * NS
    per_w = n_tasks // NW
    CH = 128
    NBUF = 4
    A = 2
    assert per_w % NBUF == 0

    idx = X.T.reshape(n_tasks, CH).astype(jnp.int32)

    mesh = plsc.VectorSubcoreMesh(core_axis_name="c", subcore_axis_name="s")

    @functools.partial(
        pl.kernel,
        mesh=mesh,
        out_type=jax.ShapeDtypeStruct((S, D // 8, RB, 8, 128), jnp.float32),
        compiler_params=pltpu.CompilerParams(
            use_tc_tiling_on_sc=False, needs_layout_passes=False
        ),
        scratch_types=[
            pltpu.VMEM((per_w, CH), jnp.int32),
            pltpu.VMEM((NBUF, CH, D), jnp.float32),
            pltpu.VMEM((NBUF, 8, 8, 128), jnp.float32),
            pltpu.SemaphoreType.DMA((NBUF,)),
            pltpu.SemaphoreType.DMA((NBUF,)),
        ],
    )
    def sc_kernel(idx_hbm, table_hbm, out_hbm, idx_v, rows_v, tbuf, gsem, ssem):
        wid = lax.axis_index("s") * NC + lax.axis_index("c")
        t0 = wid * per_w
        pltpu.sync_copy(idx_hbm.at[pl.ds(t0, per_w)], idx_v)

        iota = lax.iota(jnp.int32, 16)

        for c in range(A):
            pltpu.async_copy(
                table_hbm.at[idx_v.at[c]], rows_v.at[c], gsem.at[c]
            )

        @pl.loop(0, per_w, step=NBUF)
        def outer(j):
            for b in range(NBUF):
                jj = j + b
                nxt = jj + A
                b2 = (b + A) % NBUF

                @pl.when(nxt < per_w)
                def _fire_gather():
                    pltpu.async_copy(
                        table_hbm.at[idx_v.at[nxt]], rows_v.at[b2], gsem.at[b2]
                    )

                pltpu.make_async_copy(
                    table_hbm.at[idx_v.at[0]], rows_v.at[b], gsem.at[b]
                ).wait()

                @pl.when(jj >= NBUF)
                def _drain_store():
                    pltpu.make_async_copy(
                        tbuf.at[b],
                        out_hbm.at[0, pl.ds(0, 8), 0],
                        ssem.at[b],
                    ).wait()

                @pl.loop(0, 8)
                def _tr(jh):
                    for jl in range(8):
                        col = jnp.full((16,), 0, jnp.int32) + (jh * 8 + jl)
                        for k in range(8):
                            v = plsc.load_gather(
                                rows_v.at[b], [iota + (16 * k), col]
                            )
                            tbuf[b, jh, jl, pl.ds(16 * k, 16)] = v * SCALE

                t = t0 + jj
                s = t // RB
                rb = lax.rem(t, RB)
                pltpu.async_copy(
                    tbuf.at[b], out_hbm.at[s, pl.ds(0, 8), rb], ssem.at[b]
                )

        for b in range(NBUF):
            pltpu.make_async_copy(
                tbuf.at[b], out_hbm.at[0, pl.ds(0, 8), 0], ssem.at[b]
            ).wait()

    out5 = sc_kernel(idx, table)
    return out5.transpose(2, 4, 0, 1, 3).reshape(R, S, D)


# speedup vs baseline: 1.6431x; 1.6431x over previous
import functools

import jax
import jax.numpy as jnp
from jax import lax
from jax.experimental import pallas as pl
from jax.experimental.pallas import tpu as pltpu
from jax.experimental.pallas import tpu_sc as plsc

SCALE = 8.0


def kernel(X, table):
    R, S = X.shape
    V, D = table.shape
    RB = R // 128
    n_tasks = S * RB

    info = plsc.get_sparse_core_info()
    NC, NS = info.num_cores, info.num_subcores
    NW = NC * NS
    per_w = n_tasks // NW
    CH = 128
    NBUF = 4
    A = 2
    assert per_w % NBUF == 0

    idx = X.T.reshape(n_tasks, CH).astype(jnp.int32)

    tablep = jnp.pad(table, ((0, 0), (0, 128 - D)))

    mesh = plsc.VectorSubcoreMesh(core_axis_name="c", subcore_axis_name="s")

    @functools.partial(
        pl.kernel,
        mesh=mesh,
        out_type=jax.ShapeDtypeStruct((S, D // 8, RB, 8, 128), jnp.float32),
        compiler_params=pltpu.CompilerParams(
            use_tc_tiling_on_sc=False, needs_layout_passes=False
        ),
        scratch_types=[
            pltpu.VMEM((per_w, CH), jnp.int32),
            pltpu.VMEM((NBUF, CH, 128), jnp.float32),
            pltpu.VMEM((NBUF, 8, 8, 128), jnp.float32),
            pltpu.SemaphoreType.DMA((NBUF,)),
            pltpu.SemaphoreType.DMA((NBUF,)),
        ],
    )
    def sc_kernel(idx_hbm, table_hbm, out_hbm, idx_v, rows_v, tbuf, gsem, ssem):
        wid = lax.axis_index("s") * NC + lax.axis_index("c")
        t0 = wid * per_w
        pltpu.sync_copy(idx_hbm.at[pl.ds(t0, per_w)], idx_v)

        iota = lax.iota(jnp.int32, 16)
        rvecs = [iota + (16 * k) for k in range(8)]

        for c in range(A):
            pltpu.async_copy(
                table_hbm.at[idx_v.at[c]], rows_v.at[c], gsem.at[c]
            )

        @pl.loop(0, per_w, step=NBUF)
        def outer(j):
            for b in range(NBUF):
                jj = j + b
                nxt = jj + A
                b2 = (b + A) % NBUF

                @pl.when(nxt < per_w)
                def _fire_gather():
                    pltpu.async_copy(
                        table_hbm.at[idx_v.at[nxt]], rows_v.at[b2], gsem.at[b2]
                    )

                pltpu.make_async_copy(
                    table_hbm.at[idx_v.at[0]], rows_v.at[b], gsem.at[b]
                ).wait()

                @pl.when(jj >= NBUF)
                def _drain_store():
                    pltpu.make_async_copy(
                        tbuf.at[b],
                        out_hbm.at[0, pl.ds(0, 8), 0],
                        ssem.at[b],
                    ).wait()

                @plsc.parallel_loop(0, D, unroll=2)
                def _tr(j):
                    jh = j >> 3
                    jl = j & 7
                    col = jnp.full((16,), 0, jnp.int32) + j
                    for k in range(8):
                        v = plsc.load_gather(rows_v.at[b], [rvecs[k], col])
                        tbuf[b, jh, jl, pl.ds(16 * k, 16)] = v * SCALE

                t = t0 + jj
                s = t // RB
                rb = lax.rem(t, RB)
                pltpu.async_copy(
                    tbuf.at[b], out_hbm.at[s, pl.ds(0, 8), rb], ssem.at[b]
                )

        for b in range(NBUF):
            pltpu.make_async_copy(
                tbuf.at[b], out_hbm.at[0, pl.ds(0, 8), 0], ssem.at[b]
            ).wait()

    out5 = sc_kernel(idx, tablep)
    return out5.transpose(2, 4, 0, 1, 3).reshape(R, S, D)


# speedup vs baseline: 1.6449x; 1.0011x over previous
import functools

import jax
import jax.numpy as jnp
from jax import lax
from jax.experimental import pallas as pl
from jax.experimental.pallas import tpu as pltpu
from jax.experimental.pallas import tpu_sc as plsc

SCALE = 8.0


def kernel(X, table):
    R, S = X.shape
    V, D = table.shape
    RB = R // 128
    n_tasks = S * RB

    info = plsc.get_sparse_core_info()
    NC, NS = info.num_cores, info.num_subcores
    NW = NC * NS
    per_w = n_tasks // NW
    CH = 128
    NBUF = 4
    A = 3
    assert per_w % NBUF == 0

    idx = X.T.reshape(n_tasks, CH).astype(jnp.int32)

    tablep = jnp.pad(table, ((0, 0), (0, 128 - D)))

    mesh = plsc.VectorSubcoreMesh(core_axis_name="c", subcore_axis_name="s")

    @functools.partial(
        pl.kernel,
        mesh=mesh,
        out_type=jax.ShapeDtypeStruct((S, D // 8, RB, 8, 128), jnp.float32),
        compiler_params=pltpu.CompilerParams(
            use_tc_tiling_on_sc=False, needs_layout_passes=False
        ),
        scratch_types=[
            pltpu.VMEM((per_w, CH), jnp.int32),
            pltpu.VMEM((NBUF, CH, 128), jnp.float32),
            pltpu.VMEM((NBUF, 8, 8, 128), jnp.float32),
            pltpu.SemaphoreType.DMA((NBUF,)),
            pltpu.SemaphoreType.DMA((NBUF,)),
        ],
    )
    def sc_kernel(idx_hbm, table_hbm, out_hbm, idx_v, rows_v, tbuf, gsem, ssem):
        wid = lax.axis_index("s") * NC + lax.axis_index("c")
        t0 = wid * per_w
        pltpu.sync_copy(idx_hbm.at[pl.ds(t0, per_w)], idx_v)

        iota = lax.iota(jnp.int32, 16)
        rvecs = [iota + (16 * k) for k in range(8)]

        for c in range(A):
            pltpu.async_copy(
                table_hbm.at[idx_v.at[c]], rows_v.at[c], gsem.at[c]
            )

        @pl.loop(0, per_w, step=NBUF)
        def outer(j):
            for b in range(NBUF):
                jj = j + b
                nxt = jj + A
                b2 = (b + A) % NBUF

                @pl.when(nxt < per_w)
                def _fire_gather():
                    pltpu.async_copy(
                        table_hbm.at[idx_v.at[nxt]], rows_v.at[b2], gsem.at[b2]
                    )

                pltpu.make_async_copy(
                    table_hbm.at[idx_v.at[0]], rows_v.at[b], gsem.at[b]
                ).wait()

                @pl.when(jj >= NBUF)
                def _drain_store():
                    pltpu.make_async_copy(
                        tbuf.at[b],
                        out_hbm.at[0, pl.ds(0, 8), 0],
                        ssem.at[b],
                    ).wait()

                @plsc.parallel_loop(0, D, unroll=4)
                def _tr(j):
                    jh = j >> 3
                    jl = j & 7
                    col = jnp.full((16,), 0, jnp.int32) + j
                    for k in range(8):
                        v = plsc.load_gather(rows_v.at[b], [rvecs[k], col])
                        tbuf[b, jh, jl, pl.ds(16 * k, 16)] = v * SCALE

                t = t0 + jj
                s = t // RB
                rb = lax.rem(t, RB)
                pltpu.async_copy(
                    tbuf.at[b], out_hbm.at[s, pl.ds(0, 8), rb], ssem.at[b]
                )

        for b in range(NBUF):
            pltpu.make_async_copy(
                tbuf.at[b], out_hbm.at[0, pl.ds(0, 8), 0], ssem.at[b]
            ).wait()

    out5 = sc_kernel(idx, tablep)
    return out5.transpose(2, 4, 0, 1, 3).reshape(R, S, D)
